# baseline (speedup 1.0000x reference)
import functools

import jax
import jax.numpy as jnp
from jax.experimental import pallas as pl
from jax.experimental.pallas import tpu as pltpu
from jax.experimental.pallas import tpu_sc as plsc

N_E = 8192
E_DIM = 256
N_TOK = 16384

BT = 512
CC = 1024
BT_OH = 512
BC_OH = 4096
BT_ST = 2048
GW = 128


def _argmin_body(z_ref, emb_ref, idx_ref):
    zb = z_ref[...]
    znorm = jnp.sum(zb * zb, axis=1, keepdims=True)

    def chunk(c, carry):
        best_val, best_idx = carry
        eb = emb_ref[pl.ds(c * CC, CC), :]
        enorm = jnp.sum(eb * eb, axis=1)
        mm = jax.lax.dot_general(
            zb, eb, (((1,), (1,)), ((), ())),
            preferred_element_type=jnp.float32,
        )
        d = (znorm + enorm[None, :]) - 2.0 * mm
        lmin = jnp.min(d, axis=1, keepdims=True)
        ids = jax.lax.broadcasted_iota(jnp.int32, (BT, CC), 1) + c * CC
        lidx = jnp.min(
            jnp.where(d == lmin, ids, jnp.int32(2**30)), axis=1, keepdims=True
        )
        better = lmin < best_val
        return (jnp.where(better, lmin, best_val),
                jnp.where(better, lidx, best_idx))

    init = (jnp.full((BT, 1), jnp.inf, jnp.float32),
            jnp.zeros((BT, 1), jnp.int32))
    _, best_idx = jax.lax.fori_loop(0, N_E // CC, chunk, init)
    idx_ref[...] = best_idx


def _compute_indices(z, embedding):
    return pl.pallas_call(
        _argmin_body,
        grid=(N_TOK // BT,),
        in_specs=[
            pl.BlockSpec((BT, E_DIM), lambda i: (i, 0)),
            pl.BlockSpec((N_E, E_DIM), lambda i: (0, 0)),
        ],
        out_specs=pl.BlockSpec((BT, 1), lambda i: (i, 0)),
        out_shape=jax.ShapeDtypeStruct((N_TOK, 1), jnp.int32),
    )(z, embedding)


def _onehot_body(idx_ref, out_ref):
    j = pl.program_id(1)
    cols = jax.lax.broadcasted_iota(jnp.int32, (BT_OH, BC_OH), 1) + j * BC_OH
    out_ref[...] = (cols == idx_ref[...]).astype(jnp.float32)


def _make_onehot(idx):
    return pl.pallas_call(
        _onehot_body,
        grid=(N_TOK // BT_OH, N_E // BC_OH),
        in_specs=[pl.BlockSpec((BT_OH, 1), lambda i, j: (i, 0))],
        out_specs=pl.BlockSpec((BT_OH, BC_OH), lambda i, j: (i, j)),
        out_shape=jax.ShapeDtypeStruct((N_TOK, N_E), jnp.float32),
    )(idx)


def _gather_rows(embedding, idx_row):
    mesh = plsc.VectorSubcoreMesh(core_axis_name="c", subcore_axis_name="s")

    @functools.partial(
        pl.kernel,
        out_type=jax.ShapeDtypeStruct((N_TOK, E_DIM), jnp.float32),
        mesh=mesh,
    )
    def gather_kernel(emb_hbm, i_hbm, o_hbm):
        def body(i_vmem, o_vmem):
            pltpu.sync_copy(emb_hbm.at[i_vmem.at[0]], o_vmem)

        pltpu.emit_pipeline(
            body,
            grid=(N_TOK // GW,),
            in_specs=[pl.BlockSpec((1, GW), lambda i: (0, i))],
            out_specs=[pl.BlockSpec((GW, E_DIM), lambda i: (i, 0))],
            core_axis_name=("c", "s"),
            dimension_semantics=(pltpu.PARALLEL,),
        )(i_hbm, o_hbm)

    return gather_kernel(embedding, idx_row)


def _st_loss_body(z_ref, zq_ref, out_ref, loss_ref):
    i = pl.program_id(0)
    zb = z_ref[...]
    qb = zq_ref[...]
    diff = qb - zb
    out_ref[...] = zb + diff

    @pl.when(i == 0)
    def _():
        loss_ref[...] = jnp.zeros((1, 1), jnp.float32)

    loss_ref[...] += jnp.sum(diff * diff).reshape(1, 1)

    @pl.when(i == N_TOK // BT_ST - 1)
    def _():
        loss_ref[...] = loss_ref[...] / jnp.float32(N_TOK * E_DIM)


def _st_and_loss(z, z_q):
    return pl.pallas_call(
        _st_loss_body,
        grid=(N_TOK // BT_ST,),
        in_specs=[
            pl.BlockSpec((BT_ST, E_DIM), lambda i: (i, 0)),
            pl.BlockSpec((BT_ST, E_DIM), lambda i: (i, 0)),
        ],
        out_specs=[
            pl.BlockSpec((BT_ST, E_DIM), lambda i: (i, 0)),
            pl.BlockSpec((1, 1), lambda i: (0, 0)),
        ],
        out_shape=[
            jax.ShapeDtypeStruct((N_TOK, E_DIM), jnp.float32),
            jax.ShapeDtypeStruct((1, 1), jnp.float32),
        ],
    )(z, z_q)


def kernel(z, embedding):
    idx = _compute_indices(z, embedding)
    min_encodings = _make_onehot(idx)
    z_q = _gather_rows(embedding, idx.reshape(1, N_TOK))
    z_q_st, loss = _st_and_loss(z, z_q)
    return (loss.reshape(()), min_encodings, z_q_st, embedding, idx)


# speedup vs baseline: 1.0010x; 1.0010x over previous
import functools

import jax
import jax.numpy as jnp
from jax.experimental import pallas as pl
from jax.experimental.pallas import tpu as pltpu
from jax.experimental.pallas import tpu_sc as plsc

N_E = 8192
E_DIM = 256
N_TOK = 16384

BT = 256
CC = 1024
BT_ST = 2048
GW = 128


def _argmin_onehot_body(z_ref, emb_ref, idx_ref, oh_ref):
    zb = z_ref[...]
    znorm = jnp.sum(zb * zb, axis=1, keepdims=True)
    zm2 = zb * (-2.0)

    def chunk(c, carry):
        rmin, rc = carry
        eb = emb_ref[pl.ds(c * CC, CC), :]
        enorm = jnp.sum(eb * eb, axis=1)
        mm2 = jax.lax.dot_general(
            zm2, eb, (((1,), (1,)), ((), ())),
            preferred_element_type=jnp.float32,
        )
        d = (znorm + enorm[None, :]) + mm2
        upd = d < rmin
        return (jnp.minimum(d, rmin), jnp.where(upd, c, rc))

    init = (jnp.full((BT, CC), jnp.inf, jnp.float32),
            jnp.zeros((BT, CC), jnp.int32))
    rmin, rc = jax.lax.fori_loop(0, N_E // CC, chunk, init)

    gmin = jnp.min(rmin, axis=1, keepdims=True)
    col = jax.lax.broadcasted_iota(jnp.int32, (BT, CC), 1)
    gid = rc * CC + col
    bi = jnp.min(jnp.where(rmin == gmin, gid, jnp.int32(2**30)),
                 axis=1, keepdims=True)
    idx_ref[...] = bi
    cols = jax.lax.broadcasted_iota(jnp.int32, (BT, N_E), 1)
    oh_ref[...] = (cols == bi).astype(jnp.float32)


def _indices_and_onehot(z, embedding):
    return pl.pallas_call(
        _argmin_onehot_body,
        grid=(N_TOK // BT,),
        in_specs=[
            pl.BlockSpec((BT, E_DIM), lambda i: (i, 0)),
            pl.BlockSpec((N_E, E_DIM), lambda i: (0, 0)),
        ],
        out_specs=[
            pl.BlockSpec((BT, 1), lambda i: (i, 0)),
            pl.BlockSpec((BT, N_E), lambda i: (i, 0)),
        ],
        out_shape=[
            jax.ShapeDtypeStruct((N_TOK, 1), jnp.int32),
            jax.ShapeDtypeStruct((N_TOK, N_E), jnp.float32),
        ],
    )(z, embedding)


def _gather_rows(embedding, idx_row):
    mesh = plsc.VectorSubcoreMesh(core_axis_name="c", subcore_axis_name="s")

    @functools.partial(
        pl.kernel,
        out_type=jax.ShapeDtypeStruct((N_TOK, E_DIM), jnp.float32),
        mesh=mesh,
    )
    def gather_kernel(emb_hbm, i_hbm, o_hbm):
        def body(i_vmem, o_vmem):
            pltpu.sync_copy(emb_hbm.at[i_vmem.at[0]], o_vmem)

        pltpu.emit_pipeline(
            body,
            grid=(N_TOK // GW,),
            in_specs=[pl.BlockSpec((1, GW), lambda i: (0, i))],
            out_specs=[pl.BlockSpec((GW, E_DIM), lambda i: (i, 0))],
            core_axis_name=("c", "s"),
            dimension_semantics=(pltpu.PARALLEL,),
        )(i_hbm, o_hbm)

    return gather_kernel(embedding, idx_row)


def _st_loss_body(z_ref, zq_ref, out_ref, loss_ref):
    i = pl.program_id(0)
    zb = z_ref[...]
    qb = zq_ref[...]
    diff = qb - zb
    out_ref[...] = zb + diff

    @pl.when(i == 0)
    def _():
        loss_ref[...] = jnp.zeros((1, 1), jnp.float32)

    loss_ref[...] += jnp.sum(diff * diff).reshape(1, 1)

    @pl.when(i == N_TOK // BT_ST - 1)
    def _():
        loss_ref[...] = loss_ref[...] / jnp.float32(N_TOK * E_DIM)


def _st_and_loss(z, z_q):
    return pl.pallas_call(
        _st_loss_body,
        grid=(N_TOK // BT_ST,),
        in_specs=[
            pl.BlockSpec((BT_ST, E_DIM), lambda i: (i, 0)),
            pl.BlockSpec((BT_ST, E_DIM), lambda i: (i, 0)),
        ],
        out_specs=[
            pl.BlockSpec((BT_ST, E_DIM), lambda i: (i, 0)),
            pl.BlockSpec((1, 1), lambda i: (0, 0)),
        ],
        out_shape=[
            jax.ShapeDtypeStruct((N_TOK, E_DIM), jnp.float32),
            jax.ShapeDtypeStruct((1, 1), jnp.float32),
        ],
    )(z, z_q)


def kernel(z, embedding):
    idx, min_encodings = _indices_and_onehot(z, embedding)
    z_q = _gather_rows(embedding, idx.reshape(1, N_TOK))
    z_q_st, loss = _st_and_loss(z, z_q)
    return (loss.reshape(()), min_encodings, z_q_st, embedding, idx)


# speedup vs baseline: 1.5930x; 1.5914x over previous
import functools

import jax
import jax.numpy as jnp
from jax.experimental import pallas as pl
from jax.experimental.pallas import tpu as pltpu
from jax.experimental.pallas import tpu_sc as plsc

N_E = 8192
E_DIM = 256
N_TOK = 16384

BT = 256
CC = 1024
BT_ST = 2048
GW = 128


def _argmin_onehot_body(z_ref, emb_ref, idx_ref, oh_ref):
    zb = z_ref[...]
    znorm = jnp.sum(zb * zb, axis=1, keepdims=True)
    zm2 = zb * (-2.0)

    def chunk(c, carry):
        rmin, rc = carry
        eb = emb_ref[pl.ds(c * CC, CC), :]
        enorm = jnp.sum(eb * eb, axis=1)
        mm2 = jax.lax.dot_general(
            zm2, eb, (((1,), (1,)), ((), ())),
            preferred_element_type=jnp.float32,
        )
        d = (znorm + enorm[None, :]) + mm2
        upd = d < rmin
        return (jnp.minimum(d, rmin), jnp.where(upd, c, rc))

    carry = (jnp.full((BT, CC), jnp.inf, jnp.float32),
             jnp.zeros((BT, CC), jnp.int32))
    for c in range(N_E // CC):
        carry = chunk(c, carry)
    rmin, rc = carry

    gmin = jnp.min(rmin, axis=1, keepdims=True)
    col = jax.lax.broadcasted_iota(jnp.int32, (BT, CC), 1)
    gid = rc * CC + col
    bi = jnp.min(jnp.where(rmin == gmin, gid, jnp.int32(2**30)),
                 axis=1, keepdims=True)
    idx_ref[...] = bi
    cols = jax.lax.broadcasted_iota(jnp.int32, (BT, N_E), 1)
    oh_ref[...] = (cols == bi).astype(jnp.float32)


def _indices_and_onehot(z, embedding):
    return pl.pallas_call(
        _argmin_onehot_body,
        grid=(N_TOK // BT,),
        in_specs=[
            pl.BlockSpec((BT, E_DIM), lambda i: (i, 0)),
            pl.BlockSpec((N_E, E_DIM), lambda i: (0, 0)),
        ],
        out_specs=[
            pl.BlockSpec((BT, 1), lambda i: (i, 0)),
            pl.BlockSpec((BT, N_E), lambda i: (i, 0)),
        ],
        out_shape=[
            jax.ShapeDtypeStruct((N_TOK, 1), jnp.int32),
            jax.ShapeDtypeStruct((N_TOK, N_E), jnp.float32),
        ],
    )(z, embedding)


def _gather_rows(embedding, idx_row):
    mesh = plsc.VectorSubcoreMesh(core_axis_name="c", subcore_axis_name="s")

    @functools.partial(
        pl.kernel,
        out_type=jax.ShapeDtypeStruct((N_TOK, E_DIM), jnp.float32),
        mesh=mesh,
    )
    def gather_kernel(emb_hbm, i_hbm, o_hbm):
        def body(i_vmem, o_vmem):
            pltpu.sync_copy(emb_hbm.at[i_vmem.at[0]], o_vmem)

        pltpu.emit_pipeline(
            body,
            grid=(N_TOK // GW,),
            in_specs=[pl.BlockSpec((1, GW), lambda i: (0, i))],
            out_specs=[pl.BlockSpec((GW, E_DIM), lambda i: (i, 0))],
            core_axis_name=("c", "s"),
            dimension_semantics=(pltpu.PARALLEL,),
        )(i_hbm, o_hbm)

    return gather_kernel(embedding, idx_row)


def _st_loss_body(z_ref, zq_ref, out_ref, loss_ref):
    i = pl.program_id(0)
    zb = z_ref[...]
    qb = zq_ref[...]
    diff = qb - zb
    out_ref[...] = zb + diff

    @pl.when(i == 0)
    def _():
        loss_ref[...] = jnp.zeros((1, 1), jnp.float32)

    loss_ref[...] += jnp.sum(diff * diff).reshape(1, 1)

    @pl.when(i == N_TOK // BT_ST - 1)
    def _():
        loss_ref[...] = loss_ref[...] / jnp.float32(N_TOK * E_DIM)


def _st_and_loss(z, z_q):
    return pl.pallas_call(
        _st_loss_body,
        grid=(N_TOK // BT_ST,),
        in_specs=[
            pl.BlockSpec((BT_ST, E_DIM), lambda i: (i, 0)),
            pl.BlockSpec((BT_ST, E_DIM), lambda i: (i, 0)),
        ],
        out_specs=[
            pl.BlockSpec((BT_ST, E_DIM), lambda i: (i, 0)),
            pl.BlockSpec((1, 1), lambda i: (0, 0)),
        ],
        out_shape=[
            jax.ShapeDtypeStruct((N_TOK, E_DIM), jnp.float32),
            jax.ShapeDtypeStruct((1, 1), jnp.float32),
        ],
    )(z, z_q)


def kernel(z, embedding):
    idx, min_encodings = _indices_and_onehot(z, embedding)
    z_q = _gather_rows(embedding, idx.reshape(1, N_TOK))
    z_q_st, loss = _st_and_loss(z, z_q)
    return (loss.reshape(()), min_encodings, z_q_st, embedding, idx)


# speedup vs baseline: 1.6254x; 1.0203x over previous
import functools

import jax
import jax.numpy as jnp
from jax.experimental import pallas as pl
from jax.experimental.pallas import tpu as pltpu
from jax.experimental.pallas import tpu_sc as plsc

N_E = 8192
E_DIM = 256
N_TOK = 16384

BT = 256
CC = 1024
BT_ST = 2048
GW = 128


NB = N_TOK // BT


def _argmin_onehot_body(z_ref, emb_ref, idx_ref, oh_ref, rmin_s, rc_s):
    i = pl.program_id(0)

    @pl.when(i > 0)
    def _extract_prev():
        rmin = rmin_s[...]
        rc = rc_s[...]
        gmin = jnp.min(rmin, axis=1, keepdims=True)
        col = jax.lax.broadcasted_iota(jnp.int32, (BT, CC), 1)
        gid = rc * CC + col
        bi = jnp.min(jnp.where(rmin == gmin, gid, jnp.int32(2**30)),
                     axis=1, keepdims=True)
        idx_ref[...] = bi
        cols = jax.lax.broadcasted_iota(jnp.int32, (BT, N_E), 1)
        oh_ref[...] = (cols == bi).astype(jnp.float32)

    @pl.when(i < NB)
    def _scan_current():
        zb = z_ref[...]
        znorm = jnp.sum(zb * zb, axis=1, keepdims=True)
        zm2 = zb * (-2.0)

        def chunk(c, carry):
            rmin, rc = carry
            eb = emb_ref[pl.ds(c * CC, CC), :]
            enorm = jnp.sum(eb * eb, axis=1)
            mm2 = jax.lax.dot_general(
                zm2, eb, (((1,), (1,)), ((), ())),
                preferred_element_type=jnp.float32,
            )
            d = (znorm + enorm[None, :]) + mm2
            upd = d < rmin
            return (jnp.minimum(d, rmin), jnp.where(upd, c, rc))

        carry = (jnp.full((BT, CC), jnp.inf, jnp.float32),
                 jnp.zeros((BT, CC), jnp.int32))
        for c in range(N_E // CC):
            carry = chunk(c, carry)
        rmin_s[...] = carry[0]
        rc_s[...] = carry[1]


def _indices_and_onehot(z, embedding):
    return pl.pallas_call(
        _argmin_onehot_body,
        grid=(NB + 1,),
        in_specs=[
            pl.BlockSpec((BT, E_DIM), lambda i: (jnp.minimum(i, NB - 1), 0)),
            pl.BlockSpec((N_E, E_DIM), lambda i: (0, 0)),
        ],
        out_specs=[
            pl.BlockSpec((BT, 1), lambda i: (jnp.maximum(i - 1, 0), 0)),
            pl.BlockSpec((BT, N_E), lambda i: (jnp.maximum(i - 1, 0), 0)),
        ],
        out_shape=[
            jax.ShapeDtypeStruct((N_TOK, 1), jnp.int32),
            jax.ShapeDtypeStruct((N_TOK, N_E), jnp.float32),
        ],
        scratch_shapes=[
            pltpu.VMEM((BT, CC), jnp.float32),
            pltpu.VMEM((BT, CC), jnp.int32),
        ],
    )(z, embedding)


def _gather_rows(embedding, idx_row):
    mesh = plsc.VectorSubcoreMesh(core_axis_name="c", subcore_axis_name="s")

    @functools.partial(
        pl.kernel,
        out_type=jax.ShapeDtypeStruct((N_TOK, E_DIM), jnp.float32),
        mesh=mesh,
    )
    def gather_kernel(emb_hbm, i_hbm, o_hbm):
        def body(i_vmem, o_vmem):
            pltpu.sync_copy(emb_hbm.at[i_vmem.at[0]], o_vmem)

        pltpu.emit_pipeline(
            body,
            grid=(N_TOK // GW,),
            in_specs=[pl.BlockSpec((1, GW), lambda i: (0, i))],
            out_specs=[pl.BlockSpec((GW, E_DIM), lambda i: (i, 0))],
            core_axis_name=("c", "s"),
            dimension_semantics=(pltpu.PARALLEL,),
        )(i_hbm, o_hbm)

    return gather_kernel(embedding, idx_row)


def _st_loss_body(z_ref, zq_ref, out_ref, loss_ref):
    i = pl.program_id(0)
    zb = z_ref[...]
    qb = zq_ref[...]
    diff = qb - zb
    out_ref[...] = zb + diff

    @pl.when(i == 0)
    def _():
        loss_ref[...] = jnp.zeros((1, 1), jnp.float32)

    loss_ref[...] += jnp.sum(diff * diff).reshape(1, 1)

    @pl.when(i == N_TOK // BT_ST - 1)
    def _():
        loss_ref[...] = loss_ref[...] / jnp.float32(N_TOK * E_DIM)


def _st_and_loss(z, z_q):
    return pl.pallas_call(
        _st_loss_body,
        grid=(N_TOK // BT_ST,),
        in_specs=[
            pl.BlockSpec((BT_ST, E_DIM), lambda i: (i, 0)),
            pl.BlockSpec((BT_ST, E_DIM), lambda i: (i, 0)),
        ],
        out_specs=[
            pl.BlockSpec((BT_ST, E_DIM), lambda i: (i, 0)),
            pl.BlockSpec((1, 1), lambda i: (0, 0)),
        ],
        out_shape=[
            jax.ShapeDtypeStruct((N_TOK, E_DIM), jnp.float32),
            jax.ShapeDtypeStruct((1, 1), jnp.float32),
        ],
    )(z, z_q)


def kernel(z, embedding):
    idx, min_encodings = _indices_and_onehot(z, embedding)
    z_q = _gather_rows(embedding, idx.reshape(1, N_TOK))
    z_q_st, loss = _st_and_loss(z, z_q)
    return (loss.reshape(()), min_encodings, z_q_st, embedding, idx)


# speedup vs baseline: 1.7565x; 1.0807x over previous
import functools

import jax
import jax.numpy as jnp
from jax.experimental import pallas as pl
from jax.experimental.pallas import tpu as pltpu
from jax.experimental.pallas import tpu_sc as plsc

N_E = 8192
E_DIM = 256
N_TOK = 16384

BT = 256
CC = 1024
BT_ST = 2048
GW = 128


NB = N_TOK // BT


def _argmin_onehot_body(z_ref, emb_ref, idx_ref, oh_ref, d_s):
    zb = z_ref[...]
    znorm = jnp.sum(zb * zb, axis=1, keepdims=True)
    zm2 = zb * (-2.0)

    for c in range(N_E // CC):
        eb = emb_ref[pl.ds(c * CC, CC), :]
        enorm = jnp.sum(eb * eb, axis=1)
        mm2 = jax.lax.dot_general(
            zm2, eb, (((1,), (1,)), ((), ())),
            preferred_element_type=jnp.float32,
        )
        d_s[:, c * CC:(c + 1) * CC] = (znorm + enorm[None, :]) + mm2

    bi = jnp.argmin(d_s[...], axis=1).astype(jnp.int32)[:, None]
    idx_ref[...] = bi
    cols = jax.lax.broadcasted_iota(jnp.int32, (BT, N_E), 1)
    oh_ref[...] = (cols == bi).astype(jnp.float32)


def _indices_and_onehot(z, embedding):
    return pl.pallas_call(
        _argmin_onehot_body,
        grid=(NB,),
        in_specs=[
            pl.BlockSpec((BT, E_DIM), lambda i: (i, 0)),
            pl.BlockSpec((N_E, E_DIM), lambda i: (0, 0)),
        ],
        out_specs=[
            pl.BlockSpec((BT, 1), lambda i: (i, 0)),
            pl.BlockSpec((BT, N_E), lambda i: (i, 0)),
        ],
        out_shape=[
            jax.ShapeDtypeStruct((N_TOK, 1), jnp.int32),
            jax.ShapeDtypeStruct((N_TOK, N_E), jnp.float32),
        ],
        scratch_shapes=[
            pltpu.VMEM((BT, N_E), jnp.float32),
        ],
    )(z, embedding)


def _gather_rows(embedding, idx_row):
    mesh = plsc.VectorSubcoreMesh(core_axis_name="c", subcore_axis_name="s")

    @functools.partial(
        pl.kernel,
        out_type=jax.ShapeDtypeStruct((N_TOK, E_DIM), jnp.float32),
        mesh=mesh,
    )
    def gather_kernel(emb_hbm, i_hbm, o_hbm):
        def body(i_vmem, o_vmem):
            pltpu.sync_copy(emb_hbm.at[i_vmem.at[0]], o_vmem)

        pltpu.emit_pipeline(
            body,
            grid=(N_TOK // GW,),
            in_specs=[pl.BlockSpec((1, GW), lambda i: (0, i))],
            out_specs=[pl.BlockSpec((GW, E_DIM), lambda i: (i, 0))],
            core_axis_name=("c", "s"),
            dimension_semantics=(pltpu.PARALLEL,),
        )(i_hbm, o_hbm)

    return gather_kernel(embedding, idx_row)


def _st_loss_body(z_ref, zq_ref, out_ref, loss_ref):
    i = pl.program_id(0)
    zb = z_ref[...]
    qb = zq_ref[...]
    diff = qb - zb
    out_ref[...] = zb + diff

    @pl.when(i == 0)
    def _():
        loss_ref[...] = jnp.zeros((1, 1), jnp.float32)

    loss_ref[...] += jnp.sum(diff * diff).reshape(1, 1)

    @pl.when(i == N_TOK // BT_ST - 1)
    def _():
        loss_ref[...] = loss_ref[...] / jnp.float32(N_TOK * E_DIM)


def _st_and_loss(z, z_q):
    return pl.pallas_call(
        _st_loss_body,
        grid=(N_TOK // BT_ST,),
        in_specs=[
            pl.BlockSpec((BT_ST, E_DIM), lambda i: (i, 0)),
            pl.BlockSpec((BT_ST, E_DIM), lambda i: (i, 0)),
        ],
        out_specs=[
            pl.BlockSpec((BT_ST, E_DIM), lambda i: (i, 0)),
            pl.BlockSpec((1, 1), lambda i: (0, 0)),
        ],
        out_shape=[
            jax.ShapeDtypeStruct((N_TOK, E_DIM), jnp.float32),
            jax.ShapeDtypeStruct((1, 1), jnp.float32),
        ],
    )(z, z_q)


def kernel(z, embedding):
    idx, min_encodings = _indices_and_onehot(z, embedding)
    z_q = _gather_rows(embedding, idx.reshape(1, N_TOK))
    z_q_st, loss = _st_and_loss(z, z_q)
    return (loss.reshape(()), min_encodings, z_q_st, embedding, idx)
